# baseline (speedup 1.0000x reference)
import functools

import jax
import jax.numpy as jnp
from jax import lax
from jax.experimental import pallas as pl
from jax.experimental.pallas import tpu as pltpu
from jax.experimental.pallas import tpu_sc as plsc

E = 16
K = 2
D = 1024
F = 4096
T = 2048
BM = 256
P = 8192
MT = P // BM
BF = 256
KF = F // BF


def _router_body(x_ref, wr_ref, br_ref, idx_ref, w_ref):
    lg = jax.lax.dot_general(wr_ref[...], x_ref[...],
                             (((1,), (1,)), ((), ())),
                             preferred_element_type=jnp.float32)
    lg = lg + br_ref[...]
    ei = jax.lax.broadcasted_iota(jnp.int32, (E, T), 0)
    m1 = jnp.max(lg, axis=0, keepdims=True)
    i1 = jnp.min(jnp.where(lg == m1, ei, E), axis=0, keepdims=True)
    lg2 = jnp.where(ei == i1, -jnp.inf, lg)
    m2 = jnp.max(lg2, axis=0, keepdims=True)
    i2 = jnp.min(jnp.where(lg2 == m2, ei, E), axis=0, keepdims=True)
    b = jnp.exp(m2 - m1)
    w0 = 1.0 / (1.0 + b)
    idx_ref[0:1, :] = i1
    idx_ref[1:2, :] = i2
    w_ref[0:1, :] = w0
    w_ref[1:2, :] = 1.0 - w0


def _router(x, Wr, br):
    return pl.pallas_call(
        _router_body,
        out_shape=(jax.ShapeDtypeStruct((K, T), jnp.int32),
                   jax.ShapeDtypeStruct((K, T), jnp.float32)),
    )(x, Wr, br.reshape(E, 1))


def _ffn_body(te_ref, xg_ref, w1_ref, b1_ref, w2_ref, b2_ref, out_hbm,
              acc_ref, sem):
    kf = pl.program_id(0)
    m = pl.program_id(1)
    rows = pl.ds(m * BM, BM)
    xb = xg_ref[rows, :]
    h = jax.lax.dot_general(xb, w1_ref[0].astype(jnp.bfloat16),
                            (((1,), (0,)), ((), ())),
                            preferred_element_type=jnp.float32)
    h = h + b1_ref[0]
    h = h * jax.nn.sigmoid(h)
    contrib = jax.lax.dot_general(
        h.astype(jnp.bfloat16), w2_ref[0].astype(jnp.bfloat16),
        (((1,), (0,)), ((), ())),
        preferred_element_type=jnp.float32)

    @pl.when(kf == 0)
    def _():
        acc_ref[rows, :] = contrib + b2_ref[0]

    @pl.when(kf > 0)
    def _():
        acc_ref[rows, :] += contrib

    @pl.when(kf == KF - 1)
    def _():
        cp = pltpu.make_async_copy(acc_ref.at[rows, :], out_hbm.at[rows, :],
                                   sem)
        cp.start()
        cp.wait()


def _ffn(tile_expert, x_g, W1, b1, W2, b2):
    grid_spec = pltpu.PrefetchScalarGridSpec(
        num_scalar_prefetch=1,
        grid=(KF, MT),
        in_specs=[
            pl.BlockSpec((P, D), lambda kf, m, te: (0, 0)),
            pl.BlockSpec((1, D, BF), lambda kf, m, te: (te[m], 0, kf)),
            pl.BlockSpec((1, 1, BF), lambda kf, m, te: (te[m], 0, kf)),
            pl.BlockSpec((1, BF, D), lambda kf, m, te: (te[m], kf, 0)),
            pl.BlockSpec((1, 1, D), lambda kf, m, te: (te[m], 0, 0)),
        ],
        out_specs=pl.BlockSpec(memory_space=pl.ANY),
        scratch_shapes=[pltpu.VMEM((P, D), jnp.float32),
                        pltpu.SemaphoreType.DMA],
    )
    return pl.pallas_call(
        _ffn_body,
        grid_spec=grid_spec,
        out_shape=jax.ShapeDtypeStruct((P, D), jnp.float32),
    )(tile_expert, x_g.astype(jnp.bfloat16), W1, b1.reshape(E, 1, F), W2,
      b2.reshape(E, 1, D))


NW1 = 16
PPW = (K * T) // NW1
NCH = PPW // 16
SCCH = 32


def _dg(x, i):
    dn = lax.GatherDimensionNumbers(offset_dims=(), collapsed_slice_dims=(0,),
                                    start_index_map=(0,))
    return lax.gather(x, i[:, None], dn, (1,),
                      mode=lax.GatherScatterMode.PROMISE_IN_BOUNDS)


def _dispatch_sc(idx_flat, x):
    mesh = plsc.VectorSubcoreMesh(core_axis_name="c", subcore_axis_name="s",
                                  num_cores=1)

    @functools.partial(
        pl.kernel, mesh=mesh,
        out_type=[jax.ShapeDtypeStruct((NW1 * 16,), jnp.int32),
                  jax.ShapeDtypeStruct((K * T,), jnp.int32)],
        scratch_types=[
            pltpu.VMEM((PPW,), jnp.int32),
            pltpu.VMEM((PPW,), jnp.int32),
            pltpu.VMEM((16,), jnp.int32),
            pltpu.SemaphoreType.DMA,
        ],
        compiler_params=pltpu.CompilerParams(needs_layout_passes=False),
    )
    def k1(idx_hbm, cnt_hbm, rk_hbm, ev_ref, rk_ref, cnt_ref, sem):
        wid = lax.axis_index("s")
        jbase = wid * PPW
        lane = lax.iota(jnp.int32, 16)
        pltpu.sync_copy(idx_hbm.at[pl.ds(jbase, PPW)], ev_ref)
        cnt = jnp.zeros((16,), jnp.int32)
        for i in range(NCH):
            ev = ev_ref[pl.ds(i * 16, 16)]
            prior = _dg(cnt, ev)
            c = jnp.zeros((16,), jnp.int32)
            hist = jnp.zeros((16,), jnp.int32)
            for sh in range(1, 16):
                down = _dg(ev, jnp.abs(lane - sh))
                c = c + ((down == ev) & (lane >= sh)).astype(jnp.int32)
            for sh in range(16):
                rolled = _dg(ev, jnp.bitwise_and(lane + sh, 15))
                hist = hist + (rolled == lane).astype(jnp.int32)
            rk_ref[pl.ds(i * 16, 16)] = prior + c
            cnt = cnt + hist
        cnt_ref[...] = cnt
        pltpu.sync_copy(rk_ref, rk_hbm.at[pl.ds(jbase, PPW)])
        pltpu.sync_copy(cnt_ref, cnt_hbm.at[pl.ds(wid * 16, 16)])

    cnts, rk = k1(idx_flat)

    @functools.partial(
        pl.kernel, mesh=mesh,
        out_type=[jax.ShapeDtypeStruct((P, D), jnp.float32),
                  jax.ShapeDtypeStruct((K * T,), jnp.int32),
                  jax.ShapeDtypeStruct((MT,), jnp.int32)],
        scratch_types=[
            pltpu.VMEM((PPW,), jnp.int32),
            pltpu.VMEM((PPW,), jnp.int32),
            pltpu.VMEM((PPW,), jnp.int32),
            pltpu.VMEM((PPW // SCCH, SCCH), jnp.int32),
            pltpu.VMEM((NW1 * 16,), jnp.int32),
            pltpu.VMEM((MT,), jnp.int32),
            pltpu.VMEM((SCCH, D), jnp.float32),
            pltpu.SemaphoreType.DMA,
        ],
        compiler_params=pltpu.CompilerParams(needs_layout_passes=False),
    )
    def k2(idx_hbm, cnt_hbm, rk_hbm, x_hbm, xg_hbm, pos_hbm, te_hbm,
           ev_ref, rk_ref, posv_ref, posm_ref,
           gloc_ref, te_ref, xrow_ref, sem):
        wid = lax.axis_index("s")
        jbase = wid * PPW
        lane = lax.iota(jnp.int32, 16)
        pltpu.sync_copy(idx_hbm.at[pl.ds(jbase, PPW)], ev_ref)
        pltpu.sync_copy(rk_hbm.at[pl.ds(jbase, PPW)], rk_ref)
        pltpu.sync_copy(cnt_hbm, gloc_ref)
        pref = jnp.zeros((16,), jnp.int32)
        tot = jnp.zeros((16,), jnp.int32)
        for wp in range(NW1):
            row = gloc_ref[pl.ds(wp * 16, 16)]
            pref = pref + jnp.where(wp < wid, row, 0)
            tot = tot + row
        padded = jnp.bitwise_and(tot + (BM - 1), -BM)
        cs = jnp.cumsum(padded)
        base = (cs - padded) + pref
        for i in range(NCH):
            ev = ev_ref[pl.ds(i * 16, 16)]
            b = _dg(base, ev)
            pos = b + rk_ref[pl.ds(i * 16, 16)]
            posv_ref[pl.ds(i * 16, 16)] = pos
            posm_ref[i // 2, pl.ds((i % 2) * 16, 16)] = pos
        pltpu.sync_copy(posv_ref, pos_hbm.at[pl.ds(jbase, PPW)])
        tokb = (wid % (T // PPW)) * PPW
        for ch in range(PPW // SCCH):
            pltpu.sync_copy(x_hbm.at[pl.ds(tokb + ch * SCCH, SCCH)], xrow_ref)
            pltpu.async_copy(xrow_ref, xg_hbm.at[posm_ref.at[ch]], sem).wait()

        @pl.when(wid == 0)
        def _():
            et = cs // BM
            for half in range(MT // 16):
                tvec = lane + half * 16
                cntv = jnp.zeros((16,), jnp.int32)
                for sh in range(16):
                    rolled = _dg(et, jnp.bitwise_and(lane + sh, 15))
                    cntv = cntv + (rolled <= tvec).astype(jnp.int32)
                te_ref[pl.ds(half * 16, 16)] = jnp.minimum(cntv, E - 1)
            pltpu.sync_copy(te_ref, te_hbm)

    return k2(idx_flat, cnts, rk, x)


TPW = T // 32
CT = 16


def _combine_sc(x, y, w_flat, pos_flat):
    mesh = plsc.VectorSubcoreMesh(core_axis_name="c", subcore_axis_name="s")

    @functools.partial(
        pl.kernel, mesh=mesh,
        out_type=jax.ShapeDtypeStruct((T, D), jnp.float32),
        scratch_types=[
            pltpu.VMEM((TPW,), jnp.int32),
            pltpu.VMEM((TPW,), jnp.int32),
            pltpu.VMEM((TPW,), jnp.float32),
            pltpu.VMEM((TPW,), jnp.float32),
            pltpu.VMEM((CT, D), jnp.float32),
            pltpu.VMEM((CT, D), jnp.float32),
            pltpu.VMEM((CT, D), jnp.float32),
            pltpu.VMEM((CT, D), jnp.float32),
            pltpu.SemaphoreType.DMA,
        ],
        compiler_params=pltpu.CompilerParams(needs_layout_passes=False),
    )
    def k(x_hbm, y_hbm, w_hbm, pos_hbm, out_hbm,
          p0_ref, p1_ref, w0_ref, w1_ref, xr, y0r, y1r, outr, sem):
        wid = lax.axis_index("s") * 2 + lax.axis_index("c")
        tb = wid * TPW
        pltpu.sync_copy(pos_hbm.at[pl.ds(tb, TPW)], p0_ref)
        pltpu.sync_copy(pos_hbm.at[pl.ds(T + tb, TPW)], p1_ref)
        pltpu.sync_copy(w_hbm.at[pl.ds(tb, TPW)], w0_ref)
        pltpu.sync_copy(w_hbm.at[pl.ds(T + tb, TPW)], w1_ref)
        for c in range(TPW // CT):
            pltpu.sync_copy(x_hbm.at[pl.ds(tb + c * CT, CT)], xr)
            pltpu.async_copy(y_hbm.at[p0_ref.at[pl.ds(c * CT, CT)]],
                             y0r, sem).wait()
            pltpu.async_copy(y_hbm.at[p1_ref.at[pl.ds(c * CT, CT)]],
                             y1r, sem).wait()
            lane = lax.iota(jnp.int32, 16)
            w0v = w0_ref[pl.ds(c * CT, 16)]
            w1v = w1_ref[pl.ds(c * CT, 16)]
            for r in range(CT):
                b0 = jnp.broadcast_to(
                    jnp.sum(jnp.where(lane == r, w0v, 0.0)), (16,))
                b1 = jnp.broadcast_to(
                    jnp.sum(jnp.where(lane == r, w1v, 0.0)), (16,))

                def body(col, _):
                    sl = pl.ds(col * 16, 16)
                    outr[r, sl] = xr[r, sl] + b0 * y0r[r, sl] + b1 * y1r[r, sl]
                    return 0

                lax.fori_loop(0, D // 16, body, 0)
            pltpu.sync_copy(outr, out_hbm.at[pl.ds(tb + c * CT, CT)])

    return k(x, y, w_flat, pos_flat)


def kernel(x, W1, b1, W2, b2, Wr, br):
    idx, w = _router(x, Wr, br)
    x_g, pos, tile_expert = _dispatch_sc(idx.reshape(-1), x)
    y = _ffn(tile_expert, x_g, W1, b1, W2, b2)
    return _combine_sc(x, y, w.reshape(-1), pos)


# speedup vs baseline: 1.4092x; 1.4092x over previous
import functools

import jax
import jax.numpy as jnp
from jax import lax
from jax.experimental import pallas as pl
from jax.experimental.pallas import tpu as pltpu
from jax.experimental.pallas import tpu_sc as plsc

E = 16
K = 2
D = 1024
F = 4096
T = 2048
BM = 256
P = 8192
MT = P // BM
BF = 512
KF = F // BF


def _router_body(x_ref, wr_ref, br_ref, idx_ref, w_ref):
    lg = jax.lax.dot_general(wr_ref[...], x_ref[...],
                             (((1,), (1,)), ((), ())),
                             preferred_element_type=jnp.float32)
    lg = lg + br_ref[...]
    ei = jax.lax.broadcasted_iota(jnp.int32, (E, T), 0)
    m1 = jnp.max(lg, axis=0, keepdims=True)
    i1 = jnp.min(jnp.where(lg == m1, ei, E), axis=0, keepdims=True)
    lg2 = jnp.where(ei == i1, -jnp.inf, lg)
    m2 = jnp.max(lg2, axis=0, keepdims=True)
    i2 = jnp.min(jnp.where(lg2 == m2, ei, E), axis=0, keepdims=True)
    b = jnp.exp(m2 - m1)
    w0 = 1.0 / (1.0 + b)
    idx_ref[0:1, :] = i1
    idx_ref[1:2, :] = i2
    w_ref[0:1, :] = w0
    w_ref[1:2, :] = 1.0 - w0


def _router(x, Wr, br):
    return pl.pallas_call(
        _router_body,
        out_shape=(jax.ShapeDtypeStruct((K, T), jnp.int32),
                   jax.ShapeDtypeStruct((K, T), jnp.float32)),
    )(x, Wr, br.reshape(E, 1))


def _ffn_body(te_ref, xg_ref, w1_ref, b1_ref, w2_ref, b2_ref, out_hbm,
              acc_ref, sem):
    kf = pl.program_id(0)
    m = pl.program_id(1)
    rows = pl.ds(m * BM, BM)
    xb = xg_ref[rows, :]
    h = jax.lax.dot_general(xb, w1_ref[0].astype(jnp.bfloat16),
                            (((1,), (0,)), ((), ())),
                            preferred_element_type=jnp.float32)
    h = h + b1_ref[0]
    h = h * jax.nn.sigmoid(h)
    contrib = jax.lax.dot_general(
        h.astype(jnp.bfloat16), w2_ref[0].astype(jnp.bfloat16),
        (((1,), (0,)), ((), ())),
        preferred_element_type=jnp.float32)

    @pl.when(kf == 0)
    def _():
        acc_ref[rows, :] = contrib + b2_ref[0]

    @pl.when(kf > 0)
    def _():
        acc_ref[rows, :] += contrib

    @pl.when(kf == KF - 1)
    def _():
        cp = pltpu.make_async_copy(acc_ref.at[rows, :], out_hbm.at[rows, :],
                                   sem)
        cp.start()
        cp.wait()


def _ffn(tile_expert, x_g, W1, b1, W2, b2):
    grid_spec = pltpu.PrefetchScalarGridSpec(
        num_scalar_prefetch=1,
        grid=(KF, MT),
        in_specs=[
            pl.BlockSpec((P, D), lambda kf, m, te: (0, 0)),
            pl.BlockSpec((1, D, BF), lambda kf, m, te: (te[m], 0, kf)),
            pl.BlockSpec((1, 1, BF), lambda kf, m, te: (te[m], 0, kf)),
            pl.BlockSpec((1, BF, D), lambda kf, m, te: (te[m], kf, 0)),
            pl.BlockSpec((1, 1, D), lambda kf, m, te: (te[m], 0, 0)),
        ],
        out_specs=pl.BlockSpec(memory_space=pl.ANY),
        scratch_shapes=[pltpu.VMEM((P, D), jnp.float32),
                        pltpu.SemaphoreType.DMA],
    )
    return pl.pallas_call(
        _ffn_body,
        grid_spec=grid_spec,
        out_shape=jax.ShapeDtypeStruct((P, D), jnp.float32),
    )(tile_expert, x_g.astype(jnp.bfloat16), W1, b1.reshape(E, 1, F), W2,
      b2.reshape(E, 1, D))


NW1 = 16
PPW = (K * T) // NW1
NCH = PPW // 16
SCCH = 32


def _dg(x, i):
    dn = lax.GatherDimensionNumbers(offset_dims=(), collapsed_slice_dims=(0,),
                                    start_index_map=(0,))
    return lax.gather(x, i[:, None], dn, (1,),
                      mode=lax.GatherScatterMode.PROMISE_IN_BOUNDS)


def _dispatch_sc(idx_flat, x):
    mesh = plsc.VectorSubcoreMesh(core_axis_name="c", subcore_axis_name="s",
                                  num_cores=1)

    @functools.partial(
        pl.kernel, mesh=mesh,
        out_type=[jax.ShapeDtypeStruct((NW1 * 16,), jnp.int32),
                  jax.ShapeDtypeStruct((K * T,), jnp.int32)],
        scratch_types=[
            pltpu.VMEM((PPW,), jnp.int32),
            pltpu.VMEM((PPW,), jnp.int32),
            pltpu.VMEM((16,), jnp.int32),
            pltpu.SemaphoreType.DMA,
        ],
        compiler_params=pltpu.CompilerParams(needs_layout_passes=False),
    )
    def k1(idx_hbm, cnt_hbm, rk_hbm, ev_ref, rk_ref, cnt_ref, sem):
        wid = lax.axis_index("s")
        jbase = wid * PPW
        lane = lax.iota(jnp.int32, 16)
        pltpu.sync_copy(idx_hbm.at[pl.ds(jbase, PPW)], ev_ref)
        cnt = jnp.zeros((16,), jnp.int32)
        for i in range(NCH):
            ev = ev_ref[pl.ds(i * 16, 16)]
            prior = _dg(cnt, ev)
            c = jnp.zeros((16,), jnp.int32)
            hist = jnp.zeros((16,), jnp.int32)
            for sh in range(1, 16):
                down = _dg(ev, jnp.abs(lane - sh))
                c = c + ((down == ev) & (lane >= sh)).astype(jnp.int32)
            for sh in range(16):
                rolled = _dg(ev, jnp.bitwise_and(lane + sh, 15))
                hist = hist + (rolled == lane).astype(jnp.int32)
            rk_ref[pl.ds(i * 16, 16)] = prior + c
            cnt = cnt + hist
        cnt_ref[...] = cnt
        pltpu.sync_copy(rk_ref, rk_hbm.at[pl.ds(jbase, PPW)])
        pltpu.sync_copy(cnt_ref, cnt_hbm.at[pl.ds(wid * 16, 16)])

    cnts, rk = k1(idx_flat)

    @functools.partial(
        pl.kernel, mesh=mesh,
        out_type=[jax.ShapeDtypeStruct((P, D), jnp.float32),
                  jax.ShapeDtypeStruct((K * T,), jnp.int32),
                  jax.ShapeDtypeStruct((MT,), jnp.int32)],
        scratch_types=[
            pltpu.VMEM((PPW,), jnp.int32),
            pltpu.VMEM((PPW,), jnp.int32),
            pltpu.VMEM((PPW,), jnp.int32),
            pltpu.VMEM((PPW // SCCH, SCCH), jnp.int32),
            pltpu.VMEM((NW1 * 16,), jnp.int32),
            pltpu.VMEM((MT,), jnp.int32),
            pltpu.VMEM((SCCH, D), jnp.float32),
            pltpu.SemaphoreType.DMA,
        ],
        compiler_params=pltpu.CompilerParams(needs_layout_passes=False),
    )
    def k2(idx_hbm, cnt_hbm, rk_hbm, x_hbm, xg_hbm, pos_hbm, te_hbm,
           ev_ref, rk_ref, posv_ref, posm_ref,
           gloc_ref, te_ref, xrow_ref, sem):
        wid = lax.axis_index("s")
        jbase = wid * PPW
        lane = lax.iota(jnp.int32, 16)
        pltpu.sync_copy(idx_hbm.at[pl.ds(jbase, PPW)], ev_ref)
        pltpu.sync_copy(rk_hbm.at[pl.ds(jbase, PPW)], rk_ref)
        pltpu.sync_copy(cnt_hbm, gloc_ref)
        pref = jnp.zeros((16,), jnp.int32)
        tot = jnp.zeros((16,), jnp.int32)
        for wp in range(NW1):
            row = gloc_ref[pl.ds(wp * 16, 16)]
            pref = pref + jnp.where(wp < wid, row, 0)
            tot = tot + row
        padded = jnp.bitwise_and(tot + (BM - 1), -BM)
        cs = jnp.cumsum(padded)
        base = (cs - padded) + pref
        for i in range(NCH):
            ev = ev_ref[pl.ds(i * 16, 16)]
            b = _dg(base, ev)
            pos = b + rk_ref[pl.ds(i * 16, 16)]
            posv_ref[pl.ds(i * 16, 16)] = pos
            posm_ref[i // 2, pl.ds((i % 2) * 16, 16)] = pos
        pltpu.sync_copy(posv_ref, pos_hbm.at[pl.ds(jbase, PPW)])
        tokb = (wid % (T // PPW)) * PPW
        for ch in range(PPW // SCCH):
            pltpu.sync_copy(x_hbm.at[pl.ds(tokb + ch * SCCH, SCCH)], xrow_ref)
            pltpu.async_copy(xrow_ref, xg_hbm.at[posm_ref.at[ch]], sem).wait()

        @pl.when(wid == 0)
        def _():
            et = cs // BM
            for half in range(MT // 16):
                tvec = lane + half * 16
                cntv = jnp.zeros((16,), jnp.int32)
                for sh in range(16):
                    rolled = _dg(et, jnp.bitwise_and(lane + sh, 15))
                    cntv = cntv + (rolled <= tvec).astype(jnp.int32)
                te_ref[pl.ds(half * 16, 16)] = jnp.minimum(cntv, E - 1)
            pltpu.sync_copy(te_ref, te_hbm)

    return k2(idx_flat, cnts, rk, x)


TPW = T // 32
CT = 16


def _combine_sc(x, y, w_flat, pos_flat):
    mesh = plsc.VectorSubcoreMesh(core_axis_name="c", subcore_axis_name="s")

    @functools.partial(
        pl.kernel, mesh=mesh,
        out_type=jax.ShapeDtypeStruct((T, D), jnp.float32),
        scratch_types=[
            pltpu.VMEM((TPW,), jnp.int32),
            pltpu.VMEM((TPW,), jnp.int32),
            pltpu.VMEM((TPW,), jnp.float32),
            pltpu.VMEM((TPW,), jnp.float32),
            pltpu.VMEM((CT, D), jnp.float32),
            pltpu.VMEM((CT, D), jnp.float32),
            pltpu.VMEM((CT, D), jnp.float32),
            pltpu.VMEM((CT, D), jnp.float32),
            pltpu.SemaphoreType.DMA,
        ],
        compiler_params=pltpu.CompilerParams(needs_layout_passes=False),
    )
    def k(x_hbm, y_hbm, w_hbm, pos_hbm, out_hbm,
          p0_ref, p1_ref, w0_ref, w1_ref, xr, y0r, y1r, outr, sem):
        wid = lax.axis_index("s") * 2 + lax.axis_index("c")
        tb = wid * TPW
        pltpu.sync_copy(pos_hbm.at[pl.ds(tb, TPW)], p0_ref)
        pltpu.sync_copy(pos_hbm.at[pl.ds(T + tb, TPW)], p1_ref)
        pltpu.sync_copy(w_hbm.at[pl.ds(tb, TPW)], w0_ref)
        pltpu.sync_copy(w_hbm.at[pl.ds(T + tb, TPW)], w1_ref)
        for c in range(TPW // CT):
            pltpu.sync_copy(x_hbm.at[pl.ds(tb + c * CT, CT)], xr)
            pltpu.async_copy(y_hbm.at[p0_ref.at[pl.ds(c * CT, CT)]],
                             y0r, sem).wait()
            pltpu.async_copy(y_hbm.at[p1_ref.at[pl.ds(c * CT, CT)]],
                             y1r, sem).wait()
            lane = lax.iota(jnp.int32, 16)
            w0v = w0_ref[pl.ds(c * CT, 16)]
            w1v = w1_ref[pl.ds(c * CT, 16)]
            for r in range(CT):
                b0 = jnp.broadcast_to(
                    jnp.sum(jnp.where(lane == r, w0v, 0.0)), (16,))
                b1 = jnp.broadcast_to(
                    jnp.sum(jnp.where(lane == r, w1v, 0.0)), (16,))

                def body(col, _):
                    sl = pl.ds(col * 16, 16)
                    outr[r, sl] = xr[r, sl] + b0 * y0r[r, sl] + b1 * y1r[r, sl]
                    return 0

                lax.fori_loop(0, D // 16, body, 0)
            pltpu.sync_copy(outr, out_hbm.at[pl.ds(tb + c * CT, CT)])

    return k(x, y, w_flat, pos_flat)


def kernel(x, W1, b1, W2, b2, Wr, br):
    idx, w = _router(x, Wr, br)
    x_g, pos, tile_expert = _dispatch_sc(idx.reshape(-1), x)
    y = _ffn(tile_expert, x_g, W1, b1, W2, b2)
    return _combine_sc(x, y, w.reshape(-1), pos)


# speedup vs baseline: 1.6371x; 1.1617x over previous
import functools

import jax
import jax.numpy as jnp
from jax import lax
from jax.experimental import pallas as pl
from jax.experimental.pallas import tpu as pltpu
from jax.experimental.pallas import tpu_sc as plsc

E = 16
K = 2
D = 1024
F = 4096
T = 2048
BM = 256
P = 8192
MT = P // BM
BF = 512
KF = F // BF


def _router_body(x_ref, wr_ref, br_ref, idx_ref, w_ref):
    lg = jax.lax.dot_general(wr_ref[...], x_ref[...],
                             (((1,), (1,)), ((), ())),
                             preferred_element_type=jnp.float32)
    lg = lg + br_ref[...]
    ei = jax.lax.broadcasted_iota(jnp.int32, (E, T), 0)
    m1 = jnp.max(lg, axis=0, keepdims=True)
    i1 = jnp.min(jnp.where(lg == m1, ei, E), axis=0, keepdims=True)
    lg2 = jnp.where(ei == i1, -jnp.inf, lg)
    m2 = jnp.max(lg2, axis=0, keepdims=True)
    i2 = jnp.min(jnp.where(lg2 == m2, ei, E), axis=0, keepdims=True)
    b = jnp.exp(m2 - m1)
    w0 = 1.0 / (1.0 + b)
    idx_ref[0:1, :] = i1
    idx_ref[1:2, :] = i2
    w_ref[0:1, :] = w0
    w_ref[1:2, :] = 1.0 - w0


def _router(x, Wr, br):
    return pl.pallas_call(
        _router_body,
        out_shape=(jax.ShapeDtypeStruct((K, T), jnp.int32),
                   jax.ShapeDtypeStruct((K, T), jnp.float32)),
    )(x, Wr, br.reshape(E, 1))


def _ffn_body(te_ref, nt_ref, xg_ref, w1_ref, b1_ref, w2_ref, b2_ref, out_hbm,
              acc_ref, sem):
    kf = pl.program_id(0)
    m = pl.program_id(1)
    rows = pl.ds(m * BM, BM)

    @pl.when(m < nt_ref[0])
    def _():
        xb = xg_ref[rows, :]
        h = jax.lax.dot_general(xb, w1_ref[0].astype(jnp.bfloat16),
                                (((1,), (0,)), ((), ())),
                                preferred_element_type=jnp.float32)
        h = h + b1_ref[0]
        h = h * jax.nn.sigmoid(h)
        contrib = jax.lax.dot_general(
            h.astype(jnp.bfloat16), w2_ref[0].astype(jnp.bfloat16),
            (((1,), (0,)), ((), ())),
            preferred_element_type=jnp.float32)

        @pl.when(kf == 0)
        def _():
            acc_ref[rows, :] = contrib + b2_ref[0]

        @pl.when(kf > 0)
        def _():
            acc_ref[rows, :] += contrib

        @pl.when(kf == KF - 1)
        def _():
            cp = pltpu.make_async_copy(acc_ref.at[rows, :],
                                       out_hbm.at[rows, :], sem)
            cp.start()
            cp.wait()


def _ffn(tile_expert, ntiles, x_g, W1, b1, W2, b2):
    grid_spec = pltpu.PrefetchScalarGridSpec(
        num_scalar_prefetch=2,
        grid=(KF, MT),
        in_specs=[
            pl.BlockSpec((P, D), lambda kf, m, te, nt: (0, 0)),
            pl.BlockSpec((1, D, BF), lambda kf, m, te, nt: (te[m], 0, kf)),
            pl.BlockSpec((1, 1, BF), lambda kf, m, te, nt: (te[m], 0, kf)),
            pl.BlockSpec((1, BF, D), lambda kf, m, te, nt: (te[m], kf, 0)),
            pl.BlockSpec((1, 1, D), lambda kf, m, te, nt: (te[m], 0, 0)),
        ],
        out_specs=pl.BlockSpec(memory_space=pl.ANY),
        scratch_shapes=[pltpu.VMEM((P, D), jnp.float32),
                        pltpu.SemaphoreType.DMA],
    )
    return pl.pallas_call(
        _ffn_body,
        grid_spec=grid_spec,
        out_shape=jax.ShapeDtypeStruct((P, D), jnp.float32),
    )(tile_expert, ntiles, x_g.astype(jnp.bfloat16), W1,
      b1.reshape(E, 1, F), W2, b2.reshape(E, 1, D))


NW1 = 16
PPW = (K * T) // NW1
NCH = PPW // 16
SCCH = 32


def _dg(x, i):
    dn = lax.GatherDimensionNumbers(offset_dims=(), collapsed_slice_dims=(0,),
                                    start_index_map=(0,))
    return lax.gather(x, i[:, None], dn, (1,),
                      mode=lax.GatherScatterMode.PROMISE_IN_BOUNDS)


def _dispatch_sc(idx_flat, x):
    mesh = plsc.VectorSubcoreMesh(core_axis_name="c", subcore_axis_name="s",
                                  num_cores=1)

    @functools.partial(
        pl.kernel, mesh=mesh,
        out_type=[jax.ShapeDtypeStruct((NW1 * 16,), jnp.int32),
                  jax.ShapeDtypeStruct((K * T,), jnp.int32)],
        scratch_types=[
            pltpu.VMEM((PPW,), jnp.int32),
            pltpu.VMEM((PPW,), jnp.int32),
            pltpu.VMEM((16,), jnp.int32),
            pltpu.SemaphoreType.DMA,
        ],
        compiler_params=pltpu.CompilerParams(needs_layout_passes=False),
    )
    def k1(idx_hbm, cnt_hbm, rk_hbm, ev_ref, rk_ref, cnt_ref, sem):
        wid = lax.axis_index("s")
        jbase = wid * PPW
        lane = lax.iota(jnp.int32, 16)
        pltpu.sync_copy(idx_hbm.at[pl.ds(jbase, PPW)], ev_ref)
        cnt = jnp.zeros((16,), jnp.int32)
        for i in range(NCH):
            ev = ev_ref[pl.ds(i * 16, 16)]
            prior = _dg(cnt, ev)
            c = jnp.zeros((16,), jnp.int32)
            hist = jnp.zeros((16,), jnp.int32)
            for sh in range(1, 16):
                down = _dg(ev, jnp.abs(lane - sh))
                c = c + ((down == ev) & (lane >= sh)).astype(jnp.int32)
            for sh in range(16):
                rolled = _dg(ev, jnp.bitwise_and(lane + sh, 15))
                hist = hist + (rolled == lane).astype(jnp.int32)
            rk_ref[pl.ds(i * 16, 16)] = prior + c
            cnt = cnt + hist
        cnt_ref[...] = cnt
        pltpu.sync_copy(rk_ref, rk_hbm.at[pl.ds(jbase, PPW)])
        pltpu.sync_copy(cnt_ref, cnt_hbm.at[pl.ds(wid * 16, 16)])

    cnts, rk = k1(idx_flat)

    @functools.partial(
        pl.kernel, mesh=mesh,
        out_type=[jax.ShapeDtypeStruct((P, D), jnp.float32),
                  jax.ShapeDtypeStruct((K * T,), jnp.int32),
                  jax.ShapeDtypeStruct((MT,), jnp.int32),
                  jax.ShapeDtypeStruct((16,), jnp.int32)],
        scratch_types=[
            pltpu.VMEM((PPW,), jnp.int32),
            pltpu.VMEM((PPW,), jnp.int32),
            pltpu.VMEM((PPW,), jnp.int32),
            pltpu.VMEM((PPW // SCCH, SCCH), jnp.int32),
            pltpu.VMEM((NW1 * 16,), jnp.int32),
            pltpu.VMEM((MT,), jnp.int32),
            pltpu.VMEM((16,), jnp.int32),
            pltpu.VMEM((SCCH, D), jnp.float32),
            pltpu.SemaphoreType.DMA,
        ],
        compiler_params=pltpu.CompilerParams(needs_layout_passes=False),
    )
    def k2(idx_hbm, cnt_hbm, rk_hbm, x_hbm, xg_hbm, pos_hbm, te_hbm, nt_hbm,
           ev_ref, rk_ref, posv_ref, posm_ref,
           gloc_ref, te_ref, nt_ref, xrow_ref, sem):
        wid = lax.axis_index("s")
        jbase = wid * PPW
        lane = lax.iota(jnp.int32, 16)
        pltpu.sync_copy(idx_hbm.at[pl.ds(jbase, PPW)], ev_ref)
        pltpu.sync_copy(rk_hbm.at[pl.ds(jbase, PPW)], rk_ref)
        pltpu.sync_copy(cnt_hbm, gloc_ref)
        pref = jnp.zeros((16,), jnp.int32)
        tot = jnp.zeros((16,), jnp.int32)
        for wp in range(NW1):
            row = gloc_ref[pl.ds(wp * 16, 16)]
            pref = pref + jnp.where(wp < wid, row, 0)
            tot = tot + row
        padded = jnp.bitwise_and(tot + (BM - 1), -BM)
        cs = jnp.cumsum(padded)
        base = (cs - padded) + pref
        for i in range(NCH):
            ev = ev_ref[pl.ds(i * 16, 16)]
            b = _dg(base, ev)
            pos = b + rk_ref[pl.ds(i * 16, 16)]
            posv_ref[pl.ds(i * 16, 16)] = pos
            posm_ref[i // 2, pl.ds((i % 2) * 16, 16)] = pos
        pltpu.sync_copy(posv_ref, pos_hbm.at[pl.ds(jbase, PPW)])
        tokb = (wid % (T // PPW)) * PPW
        for ch in range(PPW // SCCH):
            pltpu.sync_copy(x_hbm.at[pl.ds(tokb + ch * SCCH, SCCH)], xrow_ref)
            pltpu.async_copy(xrow_ref, xg_hbm.at[posm_ref.at[ch]], sem).wait()

        @pl.when(wid == 0)
        def _():
            et = cs // BM
            for half in range(MT // 16):
                tvec = lane + half * 16
                cntv = jnp.zeros((16,), jnp.int32)
                for sh in range(16):
                    rolled = _dg(et, jnp.bitwise_and(lane + sh, 15))
                    cntv = cntv + (rolled <= tvec).astype(jnp.int32)
                te_ref[pl.ds(half * 16, 16)] = jnp.minimum(cntv, E - 1)
            pltpu.sync_copy(te_ref, te_hbm)
            used = jnp.sum(jnp.where(lane == E - 1, cs, 0)) // BM
            nt_ref[...] = jnp.broadcast_to(used, (16,))
            pltpu.sync_copy(nt_ref, nt_hbm)

    return k2(idx_flat, cnts, rk, x)


TPW = T // 32
CT = 16


def _combine_sc(x, y, w_flat, pos_flat):
    mesh = plsc.VectorSubcoreMesh(core_axis_name="c", subcore_axis_name="s")

    @functools.partial(
        pl.kernel, mesh=mesh,
        out_type=jax.ShapeDtypeStruct((T, D), jnp.float32),
        scratch_types=[
            pltpu.VMEM((TPW,), jnp.int32),
            pltpu.VMEM((TPW,), jnp.int32),
            pltpu.VMEM((TPW,), jnp.float32),
            pltpu.VMEM((TPW,), jnp.float32),
            pltpu.VMEM((CT, D), jnp.float32),
            pltpu.VMEM((CT, D), jnp.float32),
            pltpu.VMEM((CT, D), jnp.float32),
            pltpu.VMEM((CT, D), jnp.float32),
            pltpu.SemaphoreType.DMA,
        ],
        compiler_params=pltpu.CompilerParams(needs_layout_passes=False),
    )
    def k(x_hbm, y_hbm, w_hbm, pos_hbm, out_hbm,
          p0_ref, p1_ref, w0_ref, w1_ref, xr, y0r, y1r, outr, sem):
        wid = lax.axis_index("s") * 2 + lax.axis_index("c")
        tb = wid * TPW
        pltpu.sync_copy(pos_hbm.at[pl.ds(tb, TPW)], p0_ref)
        pltpu.sync_copy(pos_hbm.at[pl.ds(T + tb, TPW)], p1_ref)
        pltpu.sync_copy(w_hbm.at[pl.ds(tb, TPW)], w0_ref)
        pltpu.sync_copy(w_hbm.at[pl.ds(T + tb, TPW)], w1_ref)
        for c in range(TPW // CT):
            pltpu.sync_copy(x_hbm.at[pl.ds(tb + c * CT, CT)], xr)
            pltpu.async_copy(y_hbm.at[p0_ref.at[pl.ds(c * CT, CT)]],
                             y0r, sem).wait()
            pltpu.async_copy(y_hbm.at[p1_ref.at[pl.ds(c * CT, CT)]],
                             y1r, sem).wait()
            lane = lax.iota(jnp.int32, 16)
            w0v = w0_ref[pl.ds(c * CT, 16)]
            w1v = w1_ref[pl.ds(c * CT, 16)]
            for r in range(CT):
                b0 = jnp.broadcast_to(
                    jnp.sum(jnp.where(lane == r, w0v, 0.0)), (16,))
                b1 = jnp.broadcast_to(
                    jnp.sum(jnp.where(lane == r, w1v, 0.0)), (16,))

                def body(col, _):
                    sl = pl.ds(col * 16, 16)
                    outr[r, sl] = xr[r, sl] + b0 * y0r[r, sl] + b1 * y1r[r, sl]
                    return 0

                lax.fori_loop(0, D // 16, body, 0)
            pltpu.sync_copy(outr, out_hbm.at[pl.ds(tb + c * CT, CT)])

    return k(x, y, w_flat, pos_flat)


def kernel(x, W1, b1, W2, b2, Wr, br):
    idx, w = _router(x, Wr, br)
    x_g, pos, tile_expert, ntiles = _dispatch_sc(idx.reshape(-1), x)
    y = _ffn(tile_expert, ntiles, x_g, W1, b1, W2, b2)
    return _combine_sc(x, y, w.reshape(-1), pos)
